# baseline (speedup 1.0000x reference)
import functools

import jax
import jax.numpy as jnp
from jax import lax
from jax.experimental import pallas as pl
from jax.experimental.pallas import tpu as pltpu
from jax.experimental.pallas import tpu_sc as plsc

_K = 50
_N = 200
_W = 64
_B = 1024
_NW = 32
_NG = _N // 8
_MINKEY = -2147483648
_TC_B = 0


def _treemax(vs):
    vs = list(vs)
    while len(vs) > 1:
        nxt = [jnp.maximum(vs[i], vs[i + 1]) for i in range(0, len(vs) - 1, 2)]
        if len(vs) % 2:
            nxt.append(vs[-1])
        vs = nxt
    return vs[0]


def _sc_body(slabs, x_hbm, o_hbm, xv0, xv1, ov, *scr):
    wid = lax.axis_index("s") * 2 + lax.axis_index("c")
    iota = lax.iota(jnp.int32, 16)
    minkey = jnp.full((16,), _MINKEY, jnp.int32)

    ncg = _W // 16
    kvs = scr[0:ncg]
    gvs = scr[ncg:2 * ncg]
    sem0, sem1 = scr[2 * ncg], scr[2 * ncg + 1]
    sl = _N * _W

    def boff(s):
        return (_TC_B + wid * slabs + s) * sl

    def compute(s, xv):
        def build(g, _):
            for cg in range(ncg):
                ks = []
                for j in range(8):
                    n = g * 8 + j
                    v = xv[pl.ds(n * _W + cg * 16, 16)]
                    b = lax.bitcast_convert_type(v, jnp.int32)
                    t = b ^ (jnp.right_shift(b, 31) & 0x7FFFFFFF)
                    key = (t & -256) | (255 - n)
                    kvs[cg][pl.ds(n * 16, 16)] = key
                    ks.append(key)
                gvs[cg][pl.ds(g * 16, 16)] = _treemax(ks)
            return _

        lax.fori_loop(0, _NG, build, None)

        l2init = []
        for cg in range(ncg):
            for g in range(_NG, 32):
                gvs[cg][pl.ds(g * 16, 16)] = minkey
            l2init.append(tuple(
                _treemax([gvs[cg][pl.ds((sg * 8 + j) * 16, 16)]
                          for j in range(8)])
                for sg in range(4)))

        def extract(i, l2s):
            out = []
            for cg in range(ncg):
                l2 = l2s[cg]
                m = _treemax(l2)
                nn = 255 - (m & 255)
                val = plsc.load_gather(xv, [nn * _W + cg * 16 + iota])
                ov[pl.ds(i * _W + cg * 16, 16)] = val
                g = jnp.right_shift(nn, 3)
                sg = jnp.right_shift(nn, 6)
                n7 = nn & 7
                g7 = g & 7
                b0 = g * 128 + iota
                ks = [plsc.load_gather(kvs[cg], [b0 + j * 16]) for j in range(8)]
                b1 = sg * 128 + iota
                g1 = [plsc.load_gather(gvs[cg], [b1 + j * 16]) for j in range(8)]
                plsc.store_scatter(kvs[cg], [nn * 16 + iota], minkey)
                t1 = _treemax([jnp.where(n7 == j, minkey, ks[j])
                               for j in range(8)])
                plsc.store_scatter(gvs[cg], [g * 16 + iota], t1)
                t2 = _treemax([jnp.where(g7 == j, t1, g1[j])
                               for j in range(8)])
                out.append(tuple(
                    jnp.where(sg == j, t2, l2[j]) for j in range(4)))
            return tuple(out)

        lax.fori_loop(0, _K, extract, tuple(l2init))
        pltpu.sync_copy(ov, o_hbm.at[pl.ds((wid * slabs + s) * (_K * _W), _K * _W)])

    pltpu.async_copy(x_hbm.at[pl.ds(boff(0), sl)], xv0, sem0)
    npairs = slabs // 2

    def pair(t, _):
        s0 = 2 * t
        pltpu.make_async_copy(x_hbm.at[pl.ds(boff(s0), sl)], xv0, sem0).wait()
        pltpu.async_copy(x_hbm.at[pl.ds(boff(s0 + 1), sl)], xv1, sem1)
        compute(s0, xv0)
        pltpu.make_async_copy(x_hbm.at[pl.ds(boff(s0 + 1), sl)], xv1, sem1).wait()

        @pl.when(t != npairs - 1)
        def _prefetch():
            pltpu.async_copy(x_hbm.at[pl.ds(boff(s0 + 2), sl)], xv0, sem0)

        compute(s0 + 1, xv1)
        return _

    lax.fori_loop(0, npairs, pair, None)


def _make_sc_topk(nb):
    return functools.partial(
        pl.kernel,
        out_type=jax.ShapeDtypeStruct((nb * _K * _W,), jnp.float32),
        mesh=plsc.VectorSubcoreMesh(core_axis_name="c", subcore_axis_name="s"),
        compiler_params=pltpu.CompilerParams(needs_layout_passes=False),
        scratch_types=(
            [
                pltpu.VMEM((_N * _W,), jnp.float32),
                pltpu.VMEM((_N * _W,), jnp.float32),
                pltpu.VMEM((_K * _W,), jnp.float32),
            ]
            + [pltpu.VMEM((_N * 16,), jnp.int32) for _ in range(4)]
            + [pltpu.VMEM((32 * 16,), jnp.int32) for _ in range(4)]
            + [pltpu.SemaphoreType.DMA, pltpu.SemaphoreType.DMA]
        ),
    )(functools.partial(_sc_body, nb // _NW))


_sc_topk = _make_sc_topk(_B - _TC_B)


def _tc_block(x_ref, o_ref):
    x = x_ref[...]
    g2 = x.shape[0]
    g = g2 // 2
    x = x.reshape(g, 2, _N, _W)
    y = jnp.concatenate([x[:, 0], x[:, 1]], axis=2)
    iota = jax.lax.broadcasted_iota(jnp.int32, y.shape, 1).astype(jnp.float32)
    neg = jnp.float32(-jnp.inf)
    big = jnp.float32(_N)
    outs = []
    for i in range(_K):
        m = jnp.max(y, axis=1, keepdims=True)
        outs.append(m)
        if i < _K - 1:
            idx = jnp.min(jnp.where(y == m, iota, big), axis=1, keepdims=True)
            y = jnp.where(iota == idx, neg, y)
    s = jnp.concatenate(outs, axis=1)
    s = jnp.stack([s[:, :, :_W], s[:, :, _W:]], axis=1)
    o_ref[...] = s.reshape(g2, _K, _W)


def _tc_topk(x, b):
    g2 = 8
    return pl.pallas_call(
        _tc_block,
        grid=(b // g2,),
        in_specs=[pl.BlockSpec((g2, _N, _W), lambda i: (i, 0, 0))],
        out_specs=pl.BlockSpec((g2, _K, _W), lambda i: (i, 0, 0)),
        out_shape=jax.ShapeDtypeStruct((b, _K, _W), jnp.float32),
        compiler_params=pltpu.CompilerParams(
            dimension_semantics=("arbitrary",),
        ),
    )(x)


def kernel(inputs):
    x = inputs.reshape(_B, _N, _W)
    out_sc = _sc_topk(x.reshape(-1)).reshape(_B - _TC_B, _K, _W)
    if _TC_B:
        out_tc = _tc_topk(x, _TC_B)
        out_sc = jnp.concatenate([out_tc, out_sc], axis=0)
    return out_sc.reshape(_B, _K, _W, 1)


# speedup vs baseline: 1.0155x; 1.0155x over previous
import functools

import jax
import jax.numpy as jnp
from jax import lax
from jax.experimental import pallas as pl
from jax.experimental.pallas import tpu as pltpu
from jax.experimental.pallas import tpu_sc as plsc

_K = 50
_N = 200
_W = 64
_B = 1024
_NW = 32
_NG = _N // 8
_MINKEY = -2147483648
_TC_B = 0


def _treemax(vs):
    vs = list(vs)
    while len(vs) > 1:
        nxt = [jnp.maximum(vs[i], vs[i + 1]) for i in range(0, len(vs) - 1, 2)]
        if len(vs) % 2:
            nxt.append(vs[-1])
        vs = nxt
    return vs[0]


def _sc_body(slabs, x_hbm, o_hbm, xv0, xv1, ov, *scr):
    wid = lax.axis_index("s") * 2 + lax.axis_index("c")
    iota = lax.iota(jnp.int32, 16)
    minkey = jnp.full((16,), _MINKEY, jnp.int32)

    ncg = _W // 16
    kvs = scr[0:ncg]
    gvs = scr[ncg:2 * ncg]
    sem0, sem1 = scr[2 * ncg], scr[2 * ncg + 1]
    sl = _N * _W

    def boff(s):
        return (_TC_B + wid * slabs + s) * sl

    def compute(s, xv):
        def build(g, _):
            for cg in range(ncg):
                ks = []
                for j in range(8):
                    n = g * 8 + j
                    v = xv[pl.ds(n * _W + cg * 16, 16)]
                    b = lax.bitcast_convert_type(v, jnp.int32)
                    t = b ^ (jnp.right_shift(b, 31) & 0x7FFFFFFF)
                    key = (t & -256) | (255 - n)
                    kvs[cg][pl.ds(n * 16, 16)] = key
                    ks.append(key)
                gvs[cg][pl.ds(g * 16, 16)] = _treemax(ks)
            return _

        lax.fori_loop(0, _NG, build, None)

        l2init = []
        for cg in range(ncg):
            for g in range(_NG, 32):
                gvs[cg][pl.ds(g * 16, 16)] = minkey
            l2init.append(tuple(
                _treemax([gvs[cg][pl.ds((sg * 8 + j) * 16, 16)]
                          for j in range(8)])
                for sg in range(4)))

        def extract(i, l2s):
            out = []
            for cg in range(ncg):
                l2 = l2s[cg]
                m = _treemax(l2)
                nn = 255 - (m & 255)
                val = plsc.load_gather(xv, [nn * _W + cg * 16 + iota])
                ov[pl.ds(i * _W + cg * 16, 16)] = val
                plsc.store_scatter(kvs[cg], [nn * 16 + iota], minkey)
                g = jnp.right_shift(nn, 3)
                sg = jnp.right_shift(nn, 6)
                b0 = g * 128 + iota
                ks = [plsc.load_gather(kvs[cg], [b0 + j * 16]) for j in range(8)]
                plsc.store_scatter(gvs[cg], [g * 16 + iota], _treemax(ks))
                b1 = sg * 128 + iota
                g1 = [plsc.load_gather(gvs[cg], [b1 + j * 16]) for j in range(8)]
                t2 = _treemax(g1)
                out.append(tuple(
                    jnp.where(sg == j, t2, l2[j]) for j in range(4)))
            return tuple(out)

        lax.fori_loop(0, _K, extract, tuple(l2init))
        pltpu.sync_copy(ov, o_hbm.at[pl.ds((wid * slabs + s) * (_K * _W), _K * _W)])

    pltpu.async_copy(x_hbm.at[pl.ds(boff(0), sl)], xv0, sem0)
    npairs = slabs // 2

    def pair(t, _):
        s0 = 2 * t
        pltpu.make_async_copy(x_hbm.at[pl.ds(boff(s0), sl)], xv0, sem0).wait()
        pltpu.async_copy(x_hbm.at[pl.ds(boff(s0 + 1), sl)], xv1, sem1)
        compute(s0, xv0)
        pltpu.make_async_copy(x_hbm.at[pl.ds(boff(s0 + 1), sl)], xv1, sem1).wait()

        @pl.when(t != npairs - 1)
        def _prefetch():
            pltpu.async_copy(x_hbm.at[pl.ds(boff(s0 + 2), sl)], xv0, sem0)

        compute(s0 + 1, xv1)
        return _

    lax.fori_loop(0, npairs, pair, None)


def _make_sc_topk(nb):
    return functools.partial(
        pl.kernel,
        out_type=jax.ShapeDtypeStruct((nb * _K * _W,), jnp.float32),
        mesh=plsc.VectorSubcoreMesh(core_axis_name="c", subcore_axis_name="s"),
        compiler_params=pltpu.CompilerParams(needs_layout_passes=False),
        scratch_types=(
            [
                pltpu.VMEM((_N * _W,), jnp.float32),
                pltpu.VMEM((_N * _W,), jnp.float32),
                pltpu.VMEM((_K * _W,), jnp.float32),
            ]
            + [pltpu.VMEM((_N * 16,), jnp.int32) for _ in range(4)]
            + [pltpu.VMEM((32 * 16,), jnp.int32) for _ in range(4)]
            + [pltpu.SemaphoreType.DMA, pltpu.SemaphoreType.DMA]
        ),
    )(functools.partial(_sc_body, nb // _NW))


_sc_topk = _make_sc_topk(_B - _TC_B)


def _tc_block(x_ref, o_ref):
    x = x_ref[...]
    g2 = x.shape[0]
    g = g2 // 2
    x = x.reshape(g, 2, _N, _W)
    y = jnp.concatenate([x[:, 0], x[:, 1]], axis=2)
    iota = jax.lax.broadcasted_iota(jnp.int32, y.shape, 1).astype(jnp.float32)
    neg = jnp.float32(-jnp.inf)
    big = jnp.float32(_N)
    outs = []
    for i in range(_K):
        m = jnp.max(y, axis=1, keepdims=True)
        outs.append(m)
        if i < _K - 1:
            idx = jnp.min(jnp.where(y == m, iota, big), axis=1, keepdims=True)
            y = jnp.where(iota == idx, neg, y)
    s = jnp.concatenate(outs, axis=1)
    s = jnp.stack([s[:, :, :_W], s[:, :, _W:]], axis=1)
    o_ref[...] = s.reshape(g2, _K, _W)


def _tc_topk(x, b):
    g2 = 8
    return pl.pallas_call(
        _tc_block,
        grid=(b // g2,),
        in_specs=[pl.BlockSpec((g2, _N, _W), lambda i: (i, 0, 0))],
        out_specs=pl.BlockSpec((g2, _K, _W), lambda i: (i, 0, 0)),
        out_shape=jax.ShapeDtypeStruct((b, _K, _W), jnp.float32),
        compiler_params=pltpu.CompilerParams(
            dimension_semantics=("arbitrary",),
        ),
    )(x)


def kernel(inputs):
    x = inputs.reshape(_B, _N, _W)
    out_sc = _sc_topk(x.reshape(-1)).reshape(_B - _TC_B, _K, _W)
    if _TC_B:
        out_tc = _tc_topk(x, _TC_B)
        out_sc = jnp.concatenate([out_tc, out_sc], axis=0)
    return out_sc.reshape(_B, _K, _W, 1)
